# baseline (speedup 1.0000x reference)
import functools

import jax
import jax.numpy as jnp
from jax import lax
from jax.experimental import pallas as pl
from jax.experimental.pallas import tpu as pltpu
from jax.experimental.pallas import tpu_sc as plsc

_PROB_THRESHOLD = 1e-05
_NUM_OUT = 32768
_ROWS = 128
_TINY = 1.1754943508222875e-38

_KS = (1832780943, 270669613)
_KU = (64467757, 2916123636)

_BCOLS = 2048
_NB_TOTAL = _NUM_OUT // _BCOLS
_NB_SCORE = 15
_C_TC = _NB_SCORE * _BCOLS
_F_SC = _NUM_OUT - _C_TC
_NW = 32
_ROWS_PER_W = _ROWS // _NW


def _threefry_bits(k0, k1, x0, x1):
    u32 = jnp.uint32
    k0 = u32(k0)
    k1 = u32(k1)
    k2 = u32(k0 ^ k1 ^ 0x1BD11BDA)
    x0 = x0 + k0
    x1 = x1 + k1
    rot = ((13, 15, 26, 6), (17, 29, 16, 24))
    keys = ((k1, k2), (k2, k0), (k0, k1), (k1, k2), (k2, k0))
    for i in range(5):
        for r in rot[i % 2]:
            x0 = x0 + x1
            x1 = (x1 << u32(r)) | (x1 >> u32(32 - r))
            x1 = x1 ^ x0
        ka, kb = keys[i]
        x0 = x0 + ka
        x1 = x1 + kb + u32(i + 1)
    return x0 ^ x1


def _uniform_from_bits(bits, minval, maxval):
    fb = (bits >> jnp.uint32(9)) | jnp.uint32(0x3F800000)
    f = lax.bitcast_convert_type(fb, jnp.float32) - jnp.float32(1.0)
    out = f * jnp.float32(maxval - minval) + jnp.float32(minval)
    return jnp.maximum(jnp.float32(minval), out)


def _logf(u):
    i32 = jnp.int32
    bits = lax.bitcast_convert_type(u, i32)
    bits2 = bits + (i32(0x3F800000) - i32(0x3F3504F3))
    k = (bits2 >> i32(23)) - i32(127)
    mbits = (bits2 & i32(0x007FFFFF)) + i32(0x3F3504F3)
    f = lax.bitcast_convert_type(mbits, jnp.float32) - jnp.float32(1.0)
    s = f / (jnp.float32(2.0) + f)
    z = s * s
    w = z * z
    t1 = w * (jnp.float32(0.40000972152) + w * jnp.float32(0.24279078841))
    t2 = z * (jnp.float32(0.66666662693) + w * jnp.float32(0.28498786688))
    r = t2 + t1
    hfsq = jnp.float32(0.5) * f * f
    kf = k.astype(jnp.float32)
    return kf * jnp.float32(0.69313812256) + (
        kf * jnp.float32(9.0580006145e-06) + (f - (hfsq - s * (hfsq + r))))


def _tc_kernel(probs_ref, val_ref, idx_ref, xlast_ref, best_val, best_idx):
    j = pl.program_id(0)
    p = probs_ref[...]
    xl = jnp.where(p < jnp.float32(_PROB_THRESHOLD), jnp.float32(0.0), p)
    xlast_ref[...] = xl

    @pl.when(j < _NB_SCORE)
    def _():
        row = lax.broadcasted_iota(jnp.int32, (_ROWS, _BCOLS), 0)
        col = lax.broadcasted_iota(jnp.int32, (_ROWS, _BCOLS), 1)
        cnt = (row * _NUM_OUT + j * _BCOLS + col).astype(jnp.uint32)
        bits = _threefry_bits(_KS[0], _KS[1], jnp.zeros_like(cnt), cnt)
        u = _uniform_from_bits(bits, _TINY, 1.0)
        score = xl / (-jnp.log(u))

        m = jnp.max(score, axis=1, keepdims=True)
        first = jnp.min(
            jnp.where(score == m, col + j * _BCOLS, jnp.int32(_NUM_OUT)),
            axis=1, keepdims=True)

        @pl.when(j == 0)
        def _():
            best_val[...] = m
            best_idx[...] = first

        @pl.when(j > 0)
        def _():
            take = m > best_val[...]
            best_val[...] = jnp.where(take, m, best_val[...])
            best_idx[...] = jnp.where(take, first, best_idx[...])

        @pl.when(j == _NB_SCORE - 1)
        def _():
            val_ref[...] = best_val[...]
            idx_ref[...] = best_idx[...]


_SC_UNROLL = 4


def _sc_kernel(probs_hbm, val_hbm, idx_hbm, rows_v, valbuf, idxbuf):
    wid = lax.axis_index("s") * 2 + lax.axis_index("c")
    r0 = wid * _ROWS_PER_W
    pltpu.sync_copy(probs_hbm.at[pl.ds(r0, _ROWS_PER_W), pl.ds(_C_TC, _F_SC)],
                    rows_v)
    for r in range(_ROWS_PER_W):
        base = (r0 + r) * _NUM_OUT + _C_TC
        lane = lax.iota(jnp.int32, 16)
        init = ([jnp.full((16,), -1.0, jnp.float32)] * _SC_UNROLL
                + [jnp.full((16,), float(_NUM_OUT), jnp.float32)] * _SC_UNROLL)

        def body(c, carry, r=r, base=base, lane=lane):
            out = list(carry)
            for q in range(_SC_UNROLL):
                vmax, vidx = out[q], out[_SC_UNROLL + q]
                off = c * (16 * _SC_UNROLL) + q * 16
                p = rows_v[r, pl.ds(off, 16)]
                xl = jnp.where(p < jnp.float32(_PROB_THRESHOLD),
                               jnp.float32(0.0), p)
                cnt = (base + off + lane).astype(jnp.uint32)
                bits = _threefry_bits(_KS[0], _KS[1], jnp.zeros_like(cnt), cnt)
                u = _uniform_from_bits(bits, _TINY, 1.0)
                score = xl / (-_logf(u))
                take = score > vmax
                col = (_C_TC + off + lane).astype(jnp.float32)
                out[q] = jnp.where(take, score, vmax)
                out[_SC_UNROLL + q] = jnp.where(take, col, vidx)
            return tuple(out)

        res = lax.fori_loop(0, _F_SC // (16 * _SC_UNROLL), body, tuple(init))
        vmax, vidx = res[0], res[_SC_UNROLL]
        for q in range(1, _SC_UNROLL):
            take = res[q] > vmax
            tie = (res[q] == vmax) & (res[_SC_UNROLL + q] < vidx)
            upd = take | tie
            vmax = jnp.where(upd, res[q], vmax)
            vidx = jnp.where(upd, res[_SC_UNROLL + q], vidx)
        valbuf[r] = vmax
        idxbuf[r] = vidx
    pltpu.sync_copy(valbuf, val_hbm.at[pl.ds(r0, _ROWS_PER_W)])
    pltpu.sync_copy(idxbuf, idx_hbm.at[pl.ds(r0, _ROWS_PER_W)])


def _merge_kernel(tcv_ref, tci_ref, scv_ref, sci_ref, next_ref):
    vmax = scv_ref[...]
    vidx = sci_ref[...]
    sc_m = jnp.max(vmax, axis=1, keepdims=True)
    sc_idx = jnp.min(
        jnp.where(vmax == sc_m, vidx, jnp.float32(2.0 * _NUM_OUT)),
        axis=1, keepdims=True)
    take = sc_m > tcv_ref[...]
    idx = jnp.where(take, sc_idx.astype(jnp.int32), tci_ref[...])
    ucnt = lax.broadcasted_iota(jnp.int32, (_ROWS, 1), 0).astype(jnp.uint32)
    ubits = _threefry_bits(_KU[0], _KU[1], jnp.zeros_like(ucnt), ucnt)
    noise = _uniform_from_bits(ubits, 0.0, 1.0)
    nt = (idx.astype(jnp.float32) + noise) * jnp.float32(1.0 / _NUM_OUT)
    next_ref[...] = jnp.where(idx == 0, jnp.float32(0.0), nt)


@jax.jit
def kernel(probs):
    sc_val, sc_idxf = pl.kernel(
        _sc_kernel,
        out_type=(jax.ShapeDtypeStruct((_ROWS, 16), jnp.float32),
                  jax.ShapeDtypeStruct((_ROWS, 16), jnp.float32)),
        mesh=plsc.VectorSubcoreMesh(core_axis_name="c", subcore_axis_name="s"),
        scratch_types=[
            pltpu.VMEM((_ROWS_PER_W, _F_SC), jnp.float32),
            pltpu.VMEM((_ROWS_PER_W, 16), jnp.float32),
            pltpu.VMEM((_ROWS_PER_W, 16), jnp.float32),
        ],
    )(probs)

    tc_val, tc_idx, x_last = pl.pallas_call(
        _tc_kernel,
        grid=(_NB_TOTAL,),
        in_specs=[pl.BlockSpec((_ROWS, _BCOLS), lambda j: (0, j))],
        out_specs=[
            pl.BlockSpec((_ROWS, 1), lambda j: (0, 0)),
            pl.BlockSpec((_ROWS, 1), lambda j: (0, 0)),
            pl.BlockSpec((_ROWS, _BCOLS), lambda j: (0, j)),
        ],
        out_shape=[
            jax.ShapeDtypeStruct((_ROWS, 1), jnp.float32),
            jax.ShapeDtypeStruct((_ROWS, 1), jnp.int32),
            jax.ShapeDtypeStruct((_ROWS, _NUM_OUT), jnp.float32),
        ],
        scratch_shapes=[
            pltpu.VMEM((_ROWS, 1), jnp.float32),
            pltpu.VMEM((_ROWS, 1), jnp.int32),
        ],
    )(probs)

    next_token = pl.pallas_call(
        _merge_kernel,
        out_shape=jax.ShapeDtypeStruct((_ROWS, 1), jnp.float32),
    )(tc_val, tc_idx, sc_val, sc_idxf)
    return next_token, x_last


# speedup vs baseline: 1.1414x; 1.1414x over previous
import functools

import jax
import jax.numpy as jnp
from jax import lax
from jax.experimental import pallas as pl
from jax.experimental.pallas import tpu as pltpu

_PROB_THRESHOLD = 1e-05
_NUM_OUT = 32768
_ROWS = 128
_TINY = 1.1754943508222875e-38

_KS = (1832780943, 270669613)
_KU = (64467757, 2916123636)


def _threefry_bits(k0, k1, x1):
    u32 = jnp.uint32
    k0 = u32(k0)
    k1 = u32(k1)
    k2 = u32(k0 ^ k1 ^ 0x1BD11BDA)
    x0 = k0
    rot = ((13, 15, 26, 6), (17, 29, 16, 24))
    keys = ((k1, k2), (k2, k0), (k0, k1), (k1, k2), (k2, k0))
    for i in range(5):
        for r in rot[i % 2]:
            x0 = x0 + x1
            x1 = (x1 << u32(r)) | (x1 >> u32(32 - r))
            x1 = x1 ^ x0
        ka, kb = keys[i]
        x0 = x0 + ka
        x1 = x1 + kb + u32(i + 1)
    return x0 ^ x1


def _uniform_from_bits(bits, minval):
    fb = (bits >> jnp.uint32(9)) | jnp.uint32(0x3F800000)
    f = lax.bitcast_convert_type(fb, jnp.float32) - jnp.float32(1.0)
    return jnp.maximum(jnp.float32(minval), f + jnp.float32(minval))


def _decode_kernel(nblocks, bcols, probs_ref, next_ref, xlast_ref,
                   best_val, best_idx, cnt_base):
    j = pl.program_id(0)
    p = probs_ref[...]
    xl = jnp.where(p < jnp.float32(_PROB_THRESHOLD), jnp.float32(0.0), p)
    xlast_ref[...] = xl

    @pl.when(j == 0)
    def _():
        row = lax.broadcasted_iota(jnp.int32, (_ROWS, bcols), 0)
        col0 = lax.broadcasted_iota(jnp.int32, (_ROWS, bcols), 1)
        cnt_base[...] = ((row * _NUM_OUT + col0).astype(jnp.uint32)
                         + jnp.uint32(_KS[1]))

    x1 = cnt_base[...] + jnp.uint32(j * bcols)
    bits = _threefry_bits(_KS[0], _KS[1], x1)
    u = _uniform_from_bits(bits, _TINY)
    score = xl / (-jnp.log(u))

    m = jnp.max(score, axis=1, keepdims=True)
    col = lax.broadcasted_iota(jnp.int32, (_ROWS, bcols), 1)
    first = jnp.min(
        jnp.where(score == m, col + j * bcols, jnp.int32(_NUM_OUT)),
        axis=1, keepdims=True)

    @pl.when(j == 0)
    def _():
        best_val[...] = m
        best_idx[...] = first

    @pl.when(j > 0)
    def _():
        take = m > best_val[...]
        best_val[...] = jnp.where(take, m, best_val[...])
        best_idx[...] = jnp.where(take, first, best_idx[...])

    @pl.when(j == nblocks - 1)
    def _():
        idx = best_idx[...]
        ucnt = (lax.broadcasted_iota(jnp.int32, (_ROWS, 1), 0).astype(jnp.uint32)
                + jnp.uint32(_KU[1]))
        ubits = _threefry_bits(_KU[0], _KU[1], ucnt)
        noise = _uniform_from_bits(ubits, 0.0)
        nt = (idx.astype(jnp.float32) + noise) * jnp.float32(1.0 / _NUM_OUT)
        next_ref[...] = jnp.where(idx == 0, jnp.float32(0.0), nt)


@jax.jit
def kernel(probs):
    nblocks = 16
    bcols = _NUM_OUT // nblocks
    next_token, x_last = pl.pallas_call(
        functools.partial(_decode_kernel, nblocks, bcols),
        grid=(nblocks,),
        in_specs=[pl.BlockSpec((_ROWS, bcols), lambda j: (0, j))],
        out_specs=[
            pl.BlockSpec((_ROWS, 1), lambda j: (0, 0)),
            pl.BlockSpec((_ROWS, bcols), lambda j: (0, j)),
        ],
        out_shape=[
            jax.ShapeDtypeStruct((_ROWS, 1), jnp.float32),
            jax.ShapeDtypeStruct((_ROWS, _NUM_OUT), jnp.float32),
        ],
        scratch_shapes=[
            pltpu.VMEM((_ROWS, 1), jnp.float32),
            pltpu.VMEM((_ROWS, 1), jnp.int32),
            pltpu.VMEM((_ROWS, bcols), jnp.uint32),
        ],
    )(probs)
    return next_token, x_last


# speedup vs baseline: 1.1617x; 1.0178x over previous
import functools

import jax
import jax.numpy as jnp
from jax import lax
from jax.experimental import pallas as pl
from jax.experimental.pallas import tpu as pltpu

_PROB_THRESHOLD = 1e-05
_NUM_OUT = 32768
_ROWS = 128
_TINY = 1.1754943508222875e-38

_KS = (1832780943, 270669613)
_KU = (64467757, 2916123636)


def _threefry_bits(k0, k1, x1):
    u32 = jnp.uint32
    k0 = u32(k0)
    k1 = u32(k1)
    k2 = u32(k0 ^ k1 ^ 0x1BD11BDA)
    x0 = k0
    rot = ((13, 15, 26, 6), (17, 29, 16, 24))
    keys = ((k1, k2), (k2, k0), (k0, k1), (k1, k2), (k2, k0))
    for i in range(5):
        for r in rot[i % 2]:
            x0 = x0 + x1
            x1 = (x1 << u32(r)) | (x1 >> u32(32 - r))
            x1 = x1 ^ x0
        ka, kb = keys[i]
        x0 = x0 + ka
        x1 = x1 + kb + u32(i + 1)
    return x0 ^ x1


def _uniform_from_bits(bits, minval):
    fb = (bits >> jnp.uint32(9)) | jnp.uint32(0x3F800000)
    f = lax.bitcast_convert_type(fb, jnp.float32) - jnp.float32(1.0)
    return jnp.maximum(jnp.float32(minval), f + jnp.float32(minval))


def _decode_kernel(nblocks, bcols, probs_ref, next_ref, xlast_ref,
                   best_val, best_idx):
    j = pl.program_id(0)
    p = probs_ref[...]
    xl = jnp.where(p < jnp.float32(_PROB_THRESHOLD), jnp.float32(0.0), p)
    xlast_ref[...] = xl

    row = lax.broadcasted_iota(jnp.int32, (_ROWS, bcols), 0)
    col = lax.broadcasted_iota(jnp.int32, (_ROWS, bcols), 1)
    x1 = ((row * _NUM_OUT + j * bcols + col).astype(jnp.uint32)
          + jnp.uint32(_KS[1]))
    bits = _threefry_bits(_KS[0], _KS[1], x1)
    u = _uniform_from_bits(bits, _TINY)
    score = xl / (-jnp.log(u))

    m = jnp.max(score, axis=1, keepdims=True)
    first = jnp.min(
        jnp.where(score == m, col + j * bcols, jnp.int32(_NUM_OUT)),
        axis=1, keepdims=True)

    @pl.when(j == 0)
    def _():
        best_val[...] = m
        best_idx[...] = first

    @pl.when(j > 0)
    def _():
        take = m > best_val[...]
        best_val[...] = jnp.where(take, m, best_val[...])
        best_idx[...] = jnp.where(take, first, best_idx[...])

    @pl.when(j == nblocks - 1)
    def _():
        idx = best_idx[...]
        ucnt = (lax.broadcasted_iota(jnp.int32, (_ROWS, 1), 0).astype(jnp.uint32)
                + jnp.uint32(_KU[1]))
        ubits = _threefry_bits(_KU[0], _KU[1], ucnt)
        noise = _uniform_from_bits(ubits, 0.0)
        nt = (idx.astype(jnp.float32) + noise) * jnp.float32(1.0 / _NUM_OUT)
        next_ref[...] = jnp.where(idx == 0, jnp.float32(0.0), nt)


@jax.jit
def kernel(probs):
    nblocks = 16
    bcols = _NUM_OUT // nblocks
    next_token, x_last = pl.pallas_call(
        functools.partial(_decode_kernel, nblocks, bcols),
        grid=(nblocks,),
        in_specs=[pl.BlockSpec((_ROWS, bcols), lambda j: (0, j))],
        out_specs=[
            pl.BlockSpec((_ROWS, 1), lambda j: (0, 0)),
            pl.BlockSpec((_ROWS, bcols), lambda j: (0, j)),
        ],
        out_shape=[
            jax.ShapeDtypeStruct((_ROWS, 1), jnp.float32),
            jax.ShapeDtypeStruct((_ROWS, _NUM_OUT), jnp.float32),
        ],
        scratch_shapes=[
            pltpu.VMEM((_ROWS, 1), jnp.float32),
            pltpu.VMEM((_ROWS, 1), jnp.int32),
        ],
    )(probs)
    return next_token, x_last


# speedup vs baseline: 1.1726x; 1.0094x over previous
import functools

import jax
import jax.numpy as jnp
from jax import lax
from jax.experimental import pallas as pl
from jax.experimental.pallas import tpu as pltpu

_PROB_THRESHOLD = 1e-05
_NUM_OUT = 32768
_ROWS = 128
_TINY = 1.1754943508222875e-38

_KS = (1832780943, 270669613)
_KU = (64467757, 2916123636)


def _threefry_bits(k0, k1, x1):
    u32 = jnp.uint32
    k0 = u32(k0)
    k1 = u32(k1)
    k2 = u32(k0 ^ k1 ^ 0x1BD11BDA)
    x0 = k0
    rot = ((13, 15, 26, 6), (17, 29, 16, 24))
    keys = ((k1, k2), (k2, k0), (k0, k1), (k1, k2), (k2, k0))
    for i in range(5):
        for r in rot[i % 2]:
            x0 = x0 + x1
            x1 = (x1 << u32(r)) | (x1 >> u32(32 - r))
            x1 = x1 ^ x0
        ka, kb = keys[i]
        x0 = x0 + ka
        x1 = x1 + kb + u32(i + 1)
    return x0 ^ x1


def _uniform_from_bits(bits, minval):
    fb = (bits >> jnp.uint32(9)) | jnp.uint32(0x3F800000)
    f = lax.bitcast_convert_type(fb, jnp.float32) - jnp.float32(1.0)
    return f + jnp.float32(minval)


def _decode_kernel(nblocks, bcols, probs_ref, next_ref, xlast_ref,
                   best_val, best_idx):
    j = pl.program_id(0)
    p = probs_ref[...]
    xl = jnp.where(p < jnp.float32(_PROB_THRESHOLD), jnp.float32(0.0), p)
    xlast_ref[...] = xl

    row = lax.broadcasted_iota(jnp.int32, (_ROWS, bcols), 0)
    col = lax.broadcasted_iota(jnp.int32, (_ROWS, bcols), 1)
    x1 = ((row * _NUM_OUT + j * bcols + col).astype(jnp.uint32)
          + jnp.uint32(_KS[1]))
    bits = _threefry_bits(_KS[0], _KS[1], x1)
    u = _uniform_from_bits(bits, _TINY)
    score = xl / (-jnp.log(u))

    m = jnp.max(score, axis=1, keepdims=True)
    first = jnp.min(
        jnp.where(score == m, col, jnp.int32(_NUM_OUT)),
        axis=1, keepdims=True) + j * bcols

    @pl.when(j == 0)
    def _():
        best_val[...] = m
        best_idx[...] = first

    @pl.when(j > 0)
    def _():
        take = m > best_val[...]
        best_val[...] = jnp.where(take, m, best_val[...])
        best_idx[...] = jnp.where(take, first, best_idx[...])

    @pl.when(j == nblocks - 1)
    def _():
        idx = best_idx[...]
        ucnt = (lax.broadcasted_iota(jnp.int32, (_ROWS, 1), 0).astype(jnp.uint32)
                + jnp.uint32(_KU[1]))
        ubits = _threefry_bits(_KU[0], _KU[1], ucnt)
        noise = _uniform_from_bits(ubits, 0.0)
        nt = (idx.astype(jnp.float32) + noise) * jnp.float32(1.0 / _NUM_OUT)
        next_ref[...] = jnp.where(idx == 0, jnp.float32(0.0), nt)


@jax.jit
def kernel(probs):
    nblocks = 16
    bcols = _NUM_OUT // nblocks
    next_token, x_last = pl.pallas_call(
        functools.partial(_decode_kernel, nblocks, bcols),
        grid=(nblocks,),
        in_specs=[pl.BlockSpec((_ROWS, bcols), lambda j: (0, j))],
        out_specs=[
            pl.BlockSpec((_ROWS, 1), lambda j: (0, 0)),
            pl.BlockSpec((_ROWS, bcols), lambda j: (0, j)),
        ],
        out_shape=[
            jax.ShapeDtypeStruct((_ROWS, 1), jnp.float32),
            jax.ShapeDtypeStruct((_ROWS, _NUM_OUT), jnp.float32),
        ],
        scratch_shapes=[
            pltpu.VMEM((_ROWS, 1), jnp.float32),
            pltpu.VMEM((_ROWS, 1), jnp.int32),
        ],
    )(probs)
    return next_token, x_last
